# baseline (speedup 1.0000x reference)
import functools

import jax
import jax.numpy as jnp
from jax import lax
from jax.experimental import pallas as pl
from jax.experimental.pallas import tpu as pltpu
import jax.experimental.pallas.tpu_sc as plsc

N = 10000
NP = 10240
E = 160000
EPAD = 163840
EN = 170000
ENP = 172032
IN_DIM = 256
HID = 96
HEADS = 4
GATC = 16
F64 = HEADS * GATC
FP = 128
B = 50
L = 200

f32 = jnp.float32
i32 = jnp.int32

_MESH = plsc.VectorSubcoreMesh(core_axis_name="c", subcore_axis_name="s")


def _iota16():
    return lax.iota(i32, 16)


def _leaky(x):
    return jnp.where(x >= 0, x, 0.2 * x)


@functools.partial(
    pl.kernel,
    out_type=jax.ShapeDtypeStruct((NP, IN_DIM), f32),
    mesh=_MESH,
    scratch_types=[
        pltpu.VMEM((4, 80), i32),
        pltpu.VMEM((320, IN_DIM), f32),
        pltpu.SemaphoreType.DMA,
    ],
)
def _k1_embed(table_hbm, idx_hbm, out_hbm, idx_v, rows_v, sem):
    wid = lax.axis_index("s") * 2 + lax.axis_index("c")
    base = pl.multiple_of(wid * 320, 320)
    for t in range(4):
        pltpu.sync_copy(idx_hbm.at[pl.ds(base + t * 80, 80)], idx_v.at[t])
    for t in range(4):
        pltpu.async_copy(table_hbm.at[idx_v.at[t]],
                         rows_v.at[pl.ds(t * 80, 80)], sem).wait()
    pltpu.sync_copy(rows_v, out_hbm.at[pl.ds(base, 320)])


def _k2_body(x_ref, w_ref, b_ref, o_ref):
    acc = jnp.dot(x_ref[...], w_ref[...], preferred_element_type=f32)
    o_ref[...] = jnp.tanh(acc + b_ref[...])


def _k2_encode(x0, w, b):
    return pl.pallas_call(
        _k2_body,
        grid=(NP // 512,),
        in_specs=[
            pl.BlockSpec((512, IN_DIM), lambda i: (i, 0)),
            pl.BlockSpec((IN_DIM, FP), lambda i: (0, 0)),
            pl.BlockSpec((1, FP), lambda i: (0, 0)),
        ],
        out_specs=pl.BlockSpec((512, FP), lambda i: (i, 0)),
        out_shape=jax.ShapeDtypeStruct((NP, FP), f32),
    )(x0, w, b)


@functools.partial(
    pl.kernel,
    out_type=jax.ShapeDtypeStruct((2, NP, FP), f32),
    mesh=_MESH,
    scratch_types=[
        pltpu.VMEM((128,), i32),
        pltpu.VMEM((128,), i32),
        pltpu.VMEM((144,), f32),
        pltpu.VMEM((128, FP), f32),
        pltpu.VMEM((128, FP), f32),
        pltpu.VMEM_SHARED((NP, FP), f32),
        pltpu.SemaphoreType.DMA,
    ],
)
def _k3_message(x1_hbm, src_hbm, dst_hbm, attr_hbm, z128_hbm, out_hbm,
                es_v, ed_v, at_v, rows_v, sc_v, acc, sem):
    cid = lax.axis_index("c")
    sid = lax.axis_index("s")
    wid = sid * 2 + cid
    pltpu.sync_copy(z128_hbm, acc.at[pl.ds(pl.multiple_of(sid * 640, 640), 640)])
    plsc.subcore_barrier()

    def chunk(ci, _):
        e0 = pl.multiple_of(wid * 5120 + ci * 128, 128)
        pltpu.sync_copy(src_hbm.at[pl.ds(e0, 128)], es_v)
        pltpu.sync_copy(dst_hbm.at[pl.ds(e0, 128)], ed_v)
        pltpu.sync_copy(attr_hbm.at[pl.ds(e0, 128)], at_v.at[pl.ds(0, 128)])
        pltpu.async_copy(x1_hbm.at[es_v], rows_v, sem).wait()

        def edge(j, _):
            aj = at_v[pl.ds(j, 16)][0]
            bc = jnp.full((16,), aj, f32)
            for c in range(8):
                sc_v[j, pl.ds(c * 16, 16)] = rows_v[j, pl.ds(c * 16, 16)] * bc
            return 0

        lax.fori_loop(0, 128, edge, 0)
        pltpu.sync_copy(sc_v, acc.at[ed_v], add=True)
        return 0

    lax.fori_loop(0, 40, chunk, 0)
    plsc.subcore_barrier()
    r0 = pl.multiple_of(sid * 640, 640)
    pltpu.sync_copy(acc.at[pl.ds(r0, 640)], out_hbm.at[cid, pl.ds(r0, 640)])


def _k4a_body(p_ref, w_ref, bd_ref, x2_ref, hw_ref, av8_ref, mx_ref):
    x2 = p_ref[0] + p_ref[1]
    x2_ref[...] = x2
    hw = jnp.dot(x2, w_ref[...], preferred_element_type=f32)
    hw_ref[...] = hw
    av8 = lax.dot_general(bd_ref[...], hw, (((0,), (1,)), ((), ())),
                          preferred_element_type=f32)
    av8_ref[...] = av8
    cur = jnp.max(av8, axis=1, keepdims=True)

    @pl.when(pl.program_id(0) == 0)
    def _():
        mx_ref[...] = cur

    @pl.when(pl.program_id(0) != 0)
    def _():
        mx_ref[...] = jnp.maximum(mx_ref[...], cur)


def _k4a_gat1_lin(msgp, w1, bd):
    return pl.pallas_call(
        _k4a_body,
        grid=(NP // 512,),
        in_specs=[
            pl.BlockSpec((2, 512, FP), lambda i: (0, i, 0)),
            pl.BlockSpec((FP, FP), lambda i: (0, 0)),
            pl.BlockSpec((FP, 2 * HEADS), lambda i: (0, 0)),
        ],
        out_specs=[
            pl.BlockSpec((512, FP), lambda i: (i, 0)),
            pl.BlockSpec((512, FP), lambda i: (i, 0)),
            pl.BlockSpec((2 * HEADS, 512), lambda i: (0, i)),
            pl.BlockSpec((2 * HEADS, 1), lambda i: (0, 0)),
        ],
        out_shape=[
            jax.ShapeDtypeStruct((NP, FP), f32),
            jax.ShapeDtypeStruct((NP, FP), f32),
            jax.ShapeDtypeStruct((2 * HEADS, NP), f32),
            jax.ShapeDtypeStruct((2 * HEADS, 1), f32),
        ],
    )(msgp, w1, bd)


_SC4B = (
    [pltpu.VMEM((64,), f32)]
    + [pltpu.VMEM((256,), i32)] * 2
    + [pltpu.VMEM((128,), i32)] * 2
    + [pltpu.VMEM((2048,), i32)]
    + [pltpu.VMEM((2048,), f32)]
    + [pltpu.VMEM((1024,), i32)]
    + [pltpu.VMEM((1024,), f32)]
    + [pltpu.VMEM((144,), f32)] * 4
    + [pltpu.VMEM((128, FP), f32)] * 2
    + [pltpu.VMEM_SHARED((8 * NP,), f32)]
    + [pltpu.VMEM_SHARED((4 * NP,), f32)]
    + [pltpu.VMEM_SHARED((NP, FP), f32)]
    + [pltpu.SemaphoreType.DMA]
)


@functools.partial(
    pl.kernel,
    out_type=jax.ShapeDtypeStruct((2, NP, FP), f32),
    mesh=_MESH,
    scratch_types=_SC4B,
)
def _k4b_gat1_edges(a0_hbm, a1_hbm, a2_hbm, a3_hbm, a4_hbm, a5_hbm, a6_hbm,
                    a7_hbm, hw_hbm, es_hbm, ed_hbm, c1_hbm, z1_hbm, z128_hbm,
                    out_hbm, gv_v, esa_v, eda_v, esc_v, edc_v, gidx_v, gbuf_v,
                    didx_v, exb_v, cf0, cf1, cf2, cf3,
                    hwr_v, sc_v, tblA, daccA, oacc, sem):
    cid = lax.axis_index("c")
    sid = lax.axis_index("s")
    cf = [cf0, cf1, cf2, cf3]

    av_in = [a0_hbm, a1_hbm, a2_hbm, a3_hbm, a4_hbm, a5_hbm, a6_hbm, a7_hbm]
    pltpu.sync_copy(c1_hbm, gv_v)
    for h in range(8):
        @pl.when(sid == h)
        def _(h=h):
            pltpu.sync_copy(av_in[h], tblA.at[pl.ds(h * NP, NP)])
    r0 = pl.multiple_of(sid * 640, 640)
    for q in range(4):
        pltpu.sync_copy(z1_hbm, daccA.at[pl.ds(sid * 2560 + q * 640, 640)])
    pltpu.sync_copy(z128_hbm, oacc.at[pl.ds(r0, 640)])

    def zrow(j, _):
        for c in range(4):
            sc_v[j, pl.ds(F64 + c * 16, 16)] = jnp.zeros((16,), f32)
        return 0

    lax.fori_loop(0, 128, zrow, 0)
    plsc.subcore_barrier()

    def chunk_a(ci, _):
        e0 = pl.multiple_of(sid * 10752 + ci * 256, 256)
        pltpu.sync_copy(es_hbm.at[pl.ds(e0, 256)], esa_v)
        pltpu.sync_copy(ed_hbm.at[pl.ds(e0, 256)], eda_v)

        def bld(g, _):
            s16 = esa_v[pl.ds(g * 16, 16)]
            d16 = eda_v[pl.ds(g * 16, 16)]
            for h in range(4):
                gidx_v[pl.ds(h * 256 + g * 16, 16)] = h * NP + s16
                gidx_v[pl.ds((4 + h) * 256 + g * 16, 16)] = (4 + h) * NP + d16
                didx_v[pl.ds(h * 256 + g * 16, 16)] = h * NP + d16
            return 0

        lax.fori_loop(0, 16, bld, 0)
        pltpu.async_copy(tblA.at[gidx_v], gbuf_v, sem).wait()

        def group(g, _):
            for h in range(4):
                al = (gbuf_v[pl.ds(h * 256 + g * 16, 16)]
                      + gbuf_v[pl.ds((4 + h) * 256 + g * 16, 16)])
                exb_v[pl.ds(h * 256 + g * 16, 16)] = jnp.exp(
                    _leaky(al) - gv_v[pl.ds(h * 16, 16)])
            return 0

        lax.fori_loop(0, 16, group, 0)
        pltpu.async_copy(exb_v, daccA.at[didx_v], sem, add=True).wait()
        return 0

    lax.fori_loop(0, 42, chunk_a, 0)
    plsc.subcore_barrier()

    def chunk_c(ci, _):
        e0 = pl.multiple_of((sid * 2 + cid) * 5376 + ci * 128, 128)
        pltpu.sync_copy(es_hbm.at[pl.ds(e0, 128)], esc_v)
        pltpu.sync_copy(ed_hbm.at[pl.ds(e0, 128)], edc_v)
        pltpu.async_copy(hw_hbm.at[esc_v], hwr_v, sem).wait()

        def bld(g, _):
            s16 = esc_v[pl.ds(g * 16, 16)]
            d16 = edc_v[pl.ds(g * 16, 16)]
            for h in range(4):
                gidx_v[pl.ds(h * 128 + g * 16, 16)] = h * NP + s16
                gidx_v[pl.ds((4 + h) * 128 + g * 16, 16)] = (4 + h) * NP + d16
                didx_v[pl.ds(h * 128 + g * 16, 16)] = h * NP + d16
            return 0

        lax.fori_loop(0, 8, bld, 0)
        pltpu.async_copy(tblA.at[gidx_v], gbuf_v, sem).wait()
        pltpu.async_copy(daccA.at[didx_v], exb_v, sem).wait()

        def group(g, _):
            for h in range(4):
                al = (gbuf_v[pl.ds(h * 128 + g * 16, 16)]
                      + gbuf_v[pl.ds((4 + h) * 128 + g * 16, 16)])
                ex = jnp.exp(_leaky(al) - gv_v[pl.ds(h * 16, 16)])
                den = exb_v[pl.ds(h * 128 + g * 16, 16)]
                cf[h][pl.ds(g * 16, 16)] = ex / (den + 1e-16)
            return 0

        lax.fori_loop(0, 8, group, 0)

        def edge(j, _):
            for h in range(4):
                cj = cf[h][pl.ds(j, 16)][0]
                bc = jnp.full((16,), cj, f32)
                sc_v[j, pl.ds(h * 16, 16)] = hwr_v[j, pl.ds(h * 16, 16)] * bc
            return 0

        lax.fori_loop(0, 128, edge, 0)
        pltpu.sync_copy(sc_v, oacc.at[edc_v], add=True)
        return 0

    lax.fori_loop(0, 42, chunk_c, 0)
    plsc.subcore_barrier()
    r1 = pl.multiple_of(sid * 640, 640)
    pltpu.sync_copy(oacc.at[pl.ds(r1, 640)], out_hbm.at[cid, pl.ds(r1, 640)])


def _k4c_body(o1_ref, b1_ref, w2_ref, s2_ref, d2_ref, a2_ref, hw2_ref, mx_ref):
    h1 = jnp.maximum(o1_ref[0] + o1_ref[1] + b1_ref[...], 0.0)
    hw2t = lax.dot_general(w2_ref[...], h1, (((0,), (1,)), ((), ())),
                           preferred_element_type=f32)
    hw2_ref[...] = hw2t
    a2 = jnp.concatenate([hw2t * s2_ref[0, 0], hw2t * d2_ref[0, 0]], axis=0)
    a2_ref[...] = a2
    cur = jnp.max(a2, axis=1, keepdims=True)

    @pl.when(pl.program_id(0) == 0)
    def _():
        mx_ref[...] = cur

    @pl.when(pl.program_id(0) != 0)
    def _():
        mx_ref[...] = jnp.maximum(mx_ref[...], cur)


def _k4c_gat2_lin(o1p, b1, w2, s2, d2):
    return pl.pallas_call(
        _k4c_body,
        grid=(NP // 512,),
        in_specs=[
            pl.BlockSpec((2, 512, FP), lambda i: (0, i, 0)),
            pl.BlockSpec((1, FP), lambda i: (0, 0)),
            pl.BlockSpec((FP, 1), lambda i: (0, 0)),
            pl.BlockSpec((1, 1), lambda i: (0, 0)),
            pl.BlockSpec((1, 1), lambda i: (0, 0)),
        ],
        out_specs=[
            pl.BlockSpec((2, 512), lambda i: (0, i)),
            pl.BlockSpec((1, 512), lambda i: (0, i)),
            pl.BlockSpec((2, 1), lambda i: (0, 0)),
        ],
        out_shape=[
            jax.ShapeDtypeStruct((2, NP), f32),
            jax.ShapeDtypeStruct((1, NP), f32),
            jax.ShapeDtypeStruct((2, 1), f32),
        ],
    )(o1p, b1, w2, s2, d2)


_SC5B = (
    [pltpu.VMEM((16,), f32)]
    + [pltpu.VMEM((256,), i32)] * 2
    + [pltpu.VMEM((512,), i32)]
    + [pltpu.VMEM((512,), f32)]
    + [pltpu.VMEM((256,), f32)] * 3
    + [pltpu.VMEM_SHARED((2 * NP,), f32)]
    + [pltpu.VMEM_SHARED((NP,), f32)] * 3
    + [pltpu.SemaphoreType.DMA]
)


@functools.partial(
    pl.kernel,
    out_type=jax.ShapeDtypeStruct((2 * NP,), f32),
    mesh=_MESH,
    scratch_types=_SC5B,
)
def _k5b_gat2_edges(a2s_hbm, a2d_hbm, hw2_hbm, es_hbm, ed_hbm, c2_hbm, z1_hbm,
                    out_hbm, gv_v, es_v, ed_v, gidx_v, gbuf_v, exb_v, dnc_v,
                    vb_v, tblS, hw2S, dacc, oacc, sem):
    cid = lax.axis_index("c")
    sid = lax.axis_index("s")

    pltpu.sync_copy(c2_hbm, gv_v)

    @pl.when(sid == 0)
    def _():
        pltpu.sync_copy(a2s_hbm, tblS.at[pl.ds(0, NP)])

    @pl.when(sid == 1)
    def _():
        pltpu.sync_copy(a2d_hbm, tblS.at[pl.ds(NP, NP)])

    @pl.when(sid == 2)
    def _():
        pltpu.sync_copy(hw2_hbm, hw2S)

    r0 = pl.multiple_of(sid * 640, 640)
    pltpu.sync_copy(z1_hbm, dacc.at[pl.ds(r0, 640)])
    pltpu.sync_copy(z1_hbm, oacc.at[pl.ds(r0, 640)])
    plsc.subcore_barrier()

    def chunk_a(ci, _):
        e0 = pl.multiple_of(sid * 10752 + ci * 256, 256)
        pltpu.sync_copy(es_hbm.at[pl.ds(e0, 256)], es_v)
        pltpu.sync_copy(ed_hbm.at[pl.ds(e0, 256)], ed_v)

        def bld(g, _):
            sl = pl.ds(g * 16, 16)
            gidx_v[sl] = es_v[sl]
            gidx_v[pl.ds(256 + g * 16, 16)] = NP + ed_v[sl]
            return 0

        lax.fori_loop(0, 16, bld, 0)
        pltpu.async_copy(tblS.at[gidx_v], gbuf_v, sem).wait()

        def group(g, _):
            al = gbuf_v[pl.ds(g * 16, 16)] + gbuf_v[pl.ds(256 + g * 16, 16)]
            exb_v[pl.ds(g * 16, 16)] = jnp.exp(_leaky(al) - gv_v[...])
            return 0

        lax.fori_loop(0, 16, group, 0)
        pltpu.sync_copy(exb_v, dacc.at[ed_v], add=True)
        return 0

    lax.fori_loop(0, 42, chunk_a, 0)
    plsc.subcore_barrier()

    def chunk_c(ci, _):
        e0 = pl.multiple_of((sid * 2 + cid) * 5376 + ci * 256, 256)
        pltpu.sync_copy(es_hbm.at[pl.ds(e0, 256)], es_v)
        pltpu.sync_copy(ed_hbm.at[pl.ds(e0, 256)], ed_v)

        def bld(g, _):
            sl = pl.ds(g * 16, 16)
            gidx_v[sl] = es_v[sl]
            gidx_v[pl.ds(256 + g * 16, 16)] = NP + ed_v[sl]
            return 0

        lax.fori_loop(0, 16, bld, 0)
        pltpu.async_copy(tblS.at[gidx_v], gbuf_v, sem).wait()
        pltpu.async_copy(dacc.at[ed_v], dnc_v, sem).wait()
        pltpu.async_copy(hw2S.at[es_v], vb_v, sem).wait()

        def group(g, _):
            sl = pl.ds(g * 16, 16)
            al = gbuf_v[sl] + gbuf_v[pl.ds(256 + g * 16, 16)]
            ex = jnp.exp(_leaky(al) - gv_v[...])
            vb_v[sl] = vb_v[sl] * ex / (dnc_v[sl] + 1e-16)
            return 0

        lax.fori_loop(0, 16, group, 0)
        pltpu.sync_copy(vb_v, oacc.at[ed_v], add=True)
        return 0

    lax.fori_loop(0, 21, chunk_c, 0)
    plsc.subcore_barrier()
    r1 = pl.multiple_of(sid * 640, 640)
    o1 = pl.multiple_of(cid * NP + sid * 640, 128)
    pltpu.sync_copy(oacc.at[pl.ds(r1, 640)], out_hbm.at[pl.ds(o1, 640)])


def _k6_body(x2_ref, o2_ref, b2_ref, we_ref, be_ref, z_ref, xo_ref):
    att = jax.nn.sigmoid(o2_ref[0] + o2_ref[1] + b2_ref[0, 0])
    emb = jnp.tanh(jnp.dot(x2_ref[...], we_ref[...], preferred_element_type=f32)
                   + be_ref[...])
    z = att * emb
    z_ref[...] = z
    xo_ref[0, ...] = (jnp.max(z, axis=0, keepdims=True)
                      + jnp.sum(z, axis=0, keepdims=True) * (1.0 / L))


def _k6_readout(x2, o2p, b2, wemb, bemb):
    return pl.pallas_call(
        _k6_body,
        grid=(B,),
        in_specs=[
            pl.BlockSpec((L, FP), lambda i: (i, 0)),
            pl.BlockSpec((2, L, 1), lambda i: (0, i, 0)),
            pl.BlockSpec((1, 1), lambda i: (0, 0)),
            pl.BlockSpec((FP, HID), lambda i: (0, 0)),
            pl.BlockSpec((1, HID), lambda i: (0, 0)),
        ],
        out_specs=[
            pl.BlockSpec((L, HID), lambda i: (i, 0)),
            pl.BlockSpec((1, 1, HID), lambda i: (i, 0, 0)),
        ],
        out_shape=[
            jax.ShapeDtypeStruct((N, HID), f32),
            jax.ShapeDtypeStruct((B, 1, HID), f32),
        ],
    )(x2, o2p, b2, wemb, bemb)


def _k7_body(a_ref, b_ref, o_ref):
    acc = lax.dot_general(a_ref[...], b_ref[...],
                          (((1,), (1,)), ((), ())),
                          preferred_element_type=f32)
    o_ref[...] = jax.nn.sigmoid(acc)


def _k7_decode(z):
    return pl.pallas_call(
        _k7_body,
        grid=(10, 10),
        in_specs=[
            pl.BlockSpec((1024, HID), lambda i, j: (i, 0)),
            pl.BlockSpec((1024, HID), lambda i, j: (j, 0)),
        ],
        out_specs=pl.BlockSpec((1024, 1024), lambda i, j: (i, j)),
        out_shape=jax.ShapeDtypeStruct((N, N), f32),
    )(z, z)


_RB = 128
_NBLK = 79
_ACC = _RB * N + 256


@functools.partial(
    pl.kernel,
    out_type=jax.ShapeDtypeStruct((N * N,), f32),
    mesh=_MESH,
    scratch_types=[
        pltpu.VMEM((10240,), i32),
        pltpu.VMEM((10240,), f32),
        pltpu.VMEM((10240,), i32),
        pltpu.VMEM((10240,), f32),
        pltpu.VMEM_SHARED((_ACC,), f32),
        pltpu.SemaphoreType.DMA,
    ],
)
def _k8_adjacency(src_hbm, dst_hbm, attr_hbm, zbig_hbm, out_hbm,
                  fv, av, idxb, valb, acc, sem):
    cid = lax.axis_index("c")
    sid = lax.axis_index("s")
    s0 = pl.multiple_of(sid * 10240, 10240)
    pltpu.sync_copy(src_hbm.at[pl.ds(s0, 10240)], idxb)
    pltpu.sync_copy(dst_hbm.at[pl.ds(s0, 10240)], fv)
    pltpu.sync_copy(attr_hbm.at[pl.ds(s0, 10240)], av)
    z0 = pl.multiple_of(sid * 80000, 128)
    pltpu.sync_copy(zbig_hbm.at[pl.ds(z0, 80000)], acc.at[pl.ds(z0, 80000)])

    def pre(g, _):
        sl = pl.ds(g * 16, 16)
        fv[sl] = idxb[sl] * N + fv[sl]
        return 0

    lax.fori_loop(0, 640, pre, 0)
    plsc.subcore_barrier()

    def build(base):
        def group(g, _):
            sl = pl.ds(g * 16, 16)
            f16 = fv[sl]
            m = (f16 >= base * N) & (f16 < (base + _RB) * N)
            dummy = _RB * N + ((g * 16 + _iota16()) & 255)
            idxb[sl] = jnp.where(m, f16 - base * N, dummy)
            valb[sl] = jnp.where(m, av[sl], 0.0)
            return 0

        lax.fori_loop(0, 640, group, 0)

    def negate():
        def group(g, _):
            sl = pl.ds(g * 16, 16)
            valb[sl] = -valb[sl]
            return 0

        lax.fori_loop(0, 640, group, 0)

    def blk(bi, _):
        b = 2 * bi + cid
        base = b * _RB
        ok = b < _NBLK

        @pl.when(ok)
        def _():
            build(base)
            pltpu.sync_copy(valb, acc.at[idxb], add=True)

        plsc.subcore_barrier()

        @pl.when(ok)
        def _():
            row0 = base + sid * 8

            @pl.when(row0 + 8 <= N)
            def _():
                a0 = pl.multiple_of(sid * (8 * N), 128)
                o0 = pl.multiple_of(row0 * N, 128)
                pltpu.sync_copy(acc.at[pl.ds(a0, 8 * N)],
                                out_hbm.at[pl.ds(o0, 8 * N)])

        plsc.subcore_barrier()

        @pl.when(ok & (bi + 1 < 40))
        def _():
            negate()
            pltpu.sync_copy(valb, acc.at[idxb], add=True)

        return 0

    lax.fori_loop(0, 40, blk, 0)


def kernel(x_idx, edge_index, edge_attr, length, embed_table, W_enc, b_enc,
           W_gat1, att_src1, att_dst1, b_gat1, W_gat2, att_src2, att_dst2,
           b_gat2, W_emb, b_emb):
    src = edge_index[0].astype(i32)
    dst = edge_index[1].astype(i32)

    idxp = jnp.concatenate([x_idx.astype(i32), jnp.zeros((NP - N,), i32)])
    efill = jnp.arange(EPAD - E, dtype=i32) % N
    srcp = jnp.concatenate([src, efill])
    dstp = jnp.concatenate([dst, efill])
    attrp = jnp.concatenate([edge_attr.astype(f32), jnp.zeros((EPAD - E,), f32)])
    loops = jnp.arange(N, dtype=i32)
    gfill = jnp.arange(ENP - EN, dtype=i32)
    esg = jnp.concatenate([src, loops, gfill % N])
    edg = jnp.concatenate([dst, loops, N + (gfill % (NP - N))])

    z1 = jnp.zeros((640,), f32)
    z128 = jnp.zeros((640, FP), f32)
    zbig = jnp.zeros((16 * 80000,), f32)

    wencp = jnp.concatenate([W_enc, jnp.zeros((IN_DIM, FP - HID), f32)], axis=1)
    bencp = jnp.concatenate([b_enc, jnp.zeros((FP - HID,), f32)]).reshape(1, FP)
    w1p = jnp.zeros((FP, FP), f32).at[:HID, :F64].set(W_gat1)
    heads_of_col = jnp.arange(FP, dtype=i32) // GATC
    maskh = (heads_of_col[:, None] == jnp.arange(HEADS, dtype=i32)[None, :])
    attcat = jnp.concatenate(
        [jnp.concatenate([att_src1.reshape(-1), jnp.zeros((FP - F64,), f32)])[:, None],
         jnp.concatenate([att_dst1.reshape(-1), jnp.zeros((FP - F64,), f32)])[:, None]],
        axis=1)
    bd = jnp.concatenate([maskh.astype(f32) * attcat[:, :1],
                          maskh.astype(f32) * attcat[:, 1:]], axis=1)
    b1p = jnp.concatenate([b_gat1, jnp.zeros((FP - F64,), f32)]).reshape(1, FP)
    w2p = jnp.concatenate([W_gat2, jnp.zeros((FP - F64, 1), f32)], axis=0)
    wembp = jnp.concatenate([W_emb, jnp.zeros((FP - HID, HID), f32)], axis=0)

    x0 = _k1_embed(embed_table, idxp)
    x1 = _k2_encode(x0, wencp, bencp)
    msgp = _k3_message(x1, srcp, dstp, attrp, z128)
    x2, hw1, av8, mx1 = _k4a_gat1_lin(msgp, w1p, bd)

    m1 = mx1.reshape(8)
    c1 = _leaky(m1[:HEADS] + m1[HEADS:])
    c1b = jnp.broadcast_to(c1[:, None], (HEADS, 16)).reshape(64)
    o1p = _k4b_gat1_edges(av8[0], av8[1], av8[2], av8[3], av8[4], av8[5],
                          av8[6], av8[7], hw1, esg, edg, c1b, z1, z128)

    a2t, hw2t, mx2 = _k4c_gat2_lin(o1p, b1p, w2p,
                                   att_src2.reshape(1, 1), att_dst2.reshape(1, 1))
    c2 = _leaky(mx2[0, 0] + mx2[1, 0])
    c2b = jnp.full((16,), c2, f32)
    o2p = _k5b_gat2_edges(a2t[0], a2t[1], hw2t[0], esg, edg, c2b, z1)

    a_ori = _k8_adjacency(srcp, dstp, attrp, zbig)

    z, xout = _k6_readout(x2, o2p.reshape(2, NP, 1), b_gat2.reshape(1, 1),
                          wembp, b_emb.reshape(1, HID))
    a_pred = _k7_decode(z)

    return (a_pred, xout.reshape(B, HID), a_ori.reshape(N, N))


# speedup vs baseline: 1.0908x; 1.0908x over previous
import functools

import jax
import jax.numpy as jnp
from jax import lax
from jax.experimental import pallas as pl
from jax.experimental.pallas import tpu as pltpu
import jax.experimental.pallas.tpu_sc as plsc

N = 10000
NP = 10240
E = 160000
EPAD = 163840
EN = 170000
ENP = 172032
IN_DIM = 256
HID = 96
HEADS = 4
GATC = 16
F64 = HEADS * GATC
FP = 128
B = 50
L = 200

f32 = jnp.float32
i32 = jnp.int32

_MESH = plsc.VectorSubcoreMesh(core_axis_name="c", subcore_axis_name="s")


def _iota16():
    return lax.iota(i32, 16)


def _leaky(x):
    return jnp.where(x >= 0, x, 0.2 * x)


@functools.partial(
    pl.kernel,
    out_type=jax.ShapeDtypeStruct((NP, IN_DIM), f32),
    mesh=_MESH,
    scratch_types=[
        pltpu.VMEM((4, 80), i32),
        pltpu.VMEM((320, IN_DIM), f32),
        pltpu.SemaphoreType.DMA,
    ],
)
def _k1_embed(table_hbm, idx_hbm, out_hbm, idx_v, rows_v, sem):
    wid = lax.axis_index("s") * 2 + lax.axis_index("c")
    base = pl.multiple_of(wid * 320, 320)
    for t in range(4):
        pltpu.sync_copy(idx_hbm.at[pl.ds(base + t * 80, 80)], idx_v.at[t])
    for t in range(4):
        pltpu.async_copy(table_hbm.at[idx_v.at[t]],
                         rows_v.at[pl.ds(t * 80, 80)], sem).wait()
    pltpu.sync_copy(rows_v, out_hbm.at[pl.ds(base, 320)])


def _k2_body(x_ref, w_ref, b_ref, o_ref):
    acc = jnp.dot(x_ref[...], w_ref[...], preferred_element_type=f32)
    o_ref[...] = jnp.tanh(acc + b_ref[...])


def _k2_encode(x0, w, b):
    return pl.pallas_call(
        _k2_body,
        grid=(NP // 512,),
        in_specs=[
            pl.BlockSpec((512, IN_DIM), lambda i: (i, 0)),
            pl.BlockSpec((IN_DIM, FP), lambda i: (0, 0)),
            pl.BlockSpec((1, FP), lambda i: (0, 0)),
        ],
        out_specs=pl.BlockSpec((512, FP), lambda i: (i, 0)),
        out_shape=jax.ShapeDtypeStruct((NP, FP), f32),
    )(x0, w, b)


@functools.partial(
    pl.kernel,
    out_type=jax.ShapeDtypeStruct((2, NP, FP), f32),
    mesh=_MESH,
    scratch_types=[
        pltpu.VMEM((128,), i32),
        pltpu.VMEM((128,), i32),
        pltpu.VMEM((144,), f32),
        pltpu.VMEM((128, FP), f32),
        pltpu.VMEM((128, FP), f32),
        pltpu.VMEM_SHARED((NP, FP), f32),
        pltpu.SemaphoreType.DMA,
    ],
)
def _k3_message(x1_hbm, src_hbm, dst_hbm, attr_hbm, z128_hbm, out_hbm,
                es_v, ed_v, at_v, rows_v, sc_v, acc, sem):
    cid = lax.axis_index("c")
    sid = lax.axis_index("s")
    wid = sid * 2 + cid
    pltpu.sync_copy(z128_hbm, acc.at[pl.ds(pl.multiple_of(sid * 640, 640), 640)])
    plsc.subcore_barrier()

    def chunk(ci, _):
        e0 = pl.multiple_of(wid * 5120 + ci * 128, 128)
        pltpu.sync_copy(src_hbm.at[pl.ds(e0, 128)], es_v)
        pltpu.sync_copy(dst_hbm.at[pl.ds(e0, 128)], ed_v)
        pltpu.sync_copy(attr_hbm.at[pl.ds(e0, 128)], at_v.at[pl.ds(0, 128)])
        pltpu.async_copy(x1_hbm.at[es_v], rows_v, sem).wait()

        def edge(j, _):
            aj = at_v[pl.ds(j, 16)][0]
            bc = jnp.full((16,), aj, f32)
            for c in range(8):
                sc_v[j, pl.ds(c * 16, 16)] = rows_v[j, pl.ds(c * 16, 16)] * bc
            return 0

        lax.fori_loop(0, 128, edge, 0)
        pltpu.sync_copy(sc_v, acc.at[ed_v], add=True)
        return 0

    lax.fori_loop(0, 40, chunk, 0)
    plsc.subcore_barrier()
    r0 = pl.multiple_of(sid * 640, 640)
    pltpu.sync_copy(acc.at[pl.ds(r0, 640)], out_hbm.at[cid, pl.ds(r0, 640)])


def _k4a_body(p_ref, w_ref, bd_ref, x2_ref, hw_ref, av8_ref, mx_ref):
    x2 = p_ref[0] + p_ref[1]
    x2_ref[...] = x2
    hw = jnp.dot(x2, w_ref[...], preferred_element_type=f32)
    hw_ref[...] = hw
    av8 = lax.dot_general(bd_ref[...], hw, (((0,), (1,)), ((), ())),
                          preferred_element_type=f32)
    av8_ref[...] = av8
    cur = jnp.max(av8, axis=1, keepdims=True)

    @pl.when(pl.program_id(0) == 0)
    def _():
        mx_ref[...] = cur

    @pl.when(pl.program_id(0) != 0)
    def _():
        mx_ref[...] = jnp.maximum(mx_ref[...], cur)


def _k4a_gat1_lin(msgp, w1, bd):
    return pl.pallas_call(
        _k4a_body,
        grid=(NP // 512,),
        in_specs=[
            pl.BlockSpec((2, 512, FP), lambda i: (0, i, 0)),
            pl.BlockSpec((FP, FP), lambda i: (0, 0)),
            pl.BlockSpec((FP, 2 * HEADS), lambda i: (0, 0)),
        ],
        out_specs=[
            pl.BlockSpec((512, FP), lambda i: (i, 0)),
            pl.BlockSpec((512, FP), lambda i: (i, 0)),
            pl.BlockSpec((2 * HEADS, 512), lambda i: (0, i)),
            pl.BlockSpec((2 * HEADS, 1), lambda i: (0, 0)),
        ],
        out_shape=[
            jax.ShapeDtypeStruct((NP, FP), f32),
            jax.ShapeDtypeStruct((NP, FP), f32),
            jax.ShapeDtypeStruct((2 * HEADS, NP), f32),
            jax.ShapeDtypeStruct((2 * HEADS, 1), f32),
        ],
    )(msgp, w1, bd)


_SC4B = (
    [pltpu.VMEM((64,), f32)]
    + [pltpu.VMEM((256,), i32)] * 2
    + [pltpu.VMEM((128,), i32)] * 2
    + [pltpu.VMEM((2048,), i32)]
    + [pltpu.VMEM((2048,), f32)]
    + [pltpu.VMEM((1024,), i32)]
    + [pltpu.VMEM((1024,), f32)]
    + [pltpu.VMEM((144,), f32)] * 4
    + [pltpu.VMEM((128, FP), f32)] * 2
    + [pltpu.VMEM_SHARED((8 * NP,), f32)]
    + [pltpu.VMEM_SHARED((4 * NP,), f32)]
    + [pltpu.VMEM_SHARED((NP, FP), f32)]
    + [pltpu.SemaphoreType.DMA]
)


@functools.partial(
    pl.kernel,
    out_type=jax.ShapeDtypeStruct((2, NP, FP), f32),
    mesh=_MESH,
    scratch_types=_SC4B,
)
def _k4b_gat1_edges(a0_hbm, a1_hbm, a2_hbm, a3_hbm, a4_hbm, a5_hbm, a6_hbm,
                    a7_hbm, hw_hbm, es_hbm, ed_hbm, c1_hbm, z1_hbm, z128_hbm,
                    out_hbm, gv_v, esa_v, eda_v, esc_v, edc_v, gidx_v, gbuf_v,
                    didx_v, exb_v, cf0, cf1, cf2, cf3,
                    hwr_v, sc_v, tblA, daccA, oacc, sem):
    cid = lax.axis_index("c")
    sid = lax.axis_index("s")
    cf = [cf0, cf1, cf2, cf3]

    av_in = [a0_hbm, a1_hbm, a2_hbm, a3_hbm, a4_hbm, a5_hbm, a6_hbm, a7_hbm]
    pltpu.sync_copy(c1_hbm, gv_v)
    for h in range(8):
        @pl.when(sid == h)
        def _(h=h):
            pltpu.sync_copy(av_in[h], tblA.at[pl.ds(h * NP, NP)])
    r0 = pl.multiple_of(sid * 640, 640)
    for q in range(4):
        pltpu.sync_copy(z1_hbm, daccA.at[pl.ds(sid * 2560 + q * 640, 640)])
    pltpu.sync_copy(z128_hbm, oacc.at[pl.ds(r0, 640)])

    def zrow(j, _):
        for c in range(4):
            sc_v[j, pl.ds(F64 + c * 16, 16)] = jnp.zeros((16,), f32)
        return 0

    lax.fori_loop(0, 128, zrow, 0)
    plsc.subcore_barrier()

    def chunk_a(ci, _):
        e0 = pl.multiple_of(sid * 10752 + ci * 256, 256)
        pltpu.sync_copy(es_hbm.at[pl.ds(e0, 256)], esa_v)
        pltpu.sync_copy(ed_hbm.at[pl.ds(e0, 256)], eda_v)

        def bld(g, _):
            s16 = esa_v[pl.ds(g * 16, 16)]
            d16 = eda_v[pl.ds(g * 16, 16)]
            for h in range(4):
                gidx_v[pl.ds(h * 256 + g * 16, 16)] = h * NP + s16
                gidx_v[pl.ds((4 + h) * 256 + g * 16, 16)] = (4 + h) * NP + d16
                didx_v[pl.ds(h * 256 + g * 16, 16)] = h * NP + d16
            return 0

        lax.fori_loop(0, 16, bld, 0)
        pltpu.async_copy(tblA.at[gidx_v], gbuf_v, sem).wait()

        def group(g, _):
            for h in range(4):
                al = (gbuf_v[pl.ds(h * 256 + g * 16, 16)]
                      + gbuf_v[pl.ds((4 + h) * 256 + g * 16, 16)])
                exb_v[pl.ds(h * 256 + g * 16, 16)] = jnp.exp(
                    _leaky(al) - gv_v[pl.ds(h * 16, 16)])
            return 0

        lax.fori_loop(0, 16, group, 0)
        pltpu.async_copy(exb_v, daccA.at[didx_v], sem, add=True).wait()
        return 0

    lax.fori_loop(0, 42, chunk_a, 0)
    plsc.subcore_barrier()

    def chunk_c(ci, _):
        e0 = pl.multiple_of((sid * 2 + cid) * 5376 + ci * 128, 128)
        pltpu.sync_copy(es_hbm.at[pl.ds(e0, 128)], esc_v)
        pltpu.sync_copy(ed_hbm.at[pl.ds(e0, 128)], edc_v)
        pltpu.async_copy(hw_hbm.at[esc_v], hwr_v, sem).wait()

        def bld(g, _):
            s16 = esc_v[pl.ds(g * 16, 16)]
            d16 = edc_v[pl.ds(g * 16, 16)]
            for h in range(4):
                gidx_v[pl.ds(h * 128 + g * 16, 16)] = h * NP + s16
                gidx_v[pl.ds((4 + h) * 128 + g * 16, 16)] = (4 + h) * NP + d16
                didx_v[pl.ds(h * 128 + g * 16, 16)] = h * NP + d16
            return 0

        lax.fori_loop(0, 8, bld, 0)
        pltpu.async_copy(tblA.at[gidx_v], gbuf_v, sem).wait()
        pltpu.async_copy(daccA.at[didx_v], exb_v, sem).wait()

        def group(g, _):
            for h in range(4):
                al = (gbuf_v[pl.ds(h * 128 + g * 16, 16)]
                      + gbuf_v[pl.ds((4 + h) * 128 + g * 16, 16)])
                ex = jnp.exp(_leaky(al) - gv_v[pl.ds(h * 16, 16)])
                den = exb_v[pl.ds(h * 128 + g * 16, 16)]
                cf[h][pl.ds(g * 16, 16)] = ex / (den + 1e-16)
            return 0

        lax.fori_loop(0, 8, group, 0)

        def edge(j, _):
            for h in range(4):
                cj = cf[h][pl.ds(j, 16)][0]
                bc = jnp.full((16,), cj, f32)
                sc_v[j, pl.ds(h * 16, 16)] = hwr_v[j, pl.ds(h * 16, 16)] * bc
            return 0

        lax.fori_loop(0, 128, edge, 0)
        pltpu.sync_copy(sc_v, oacc.at[edc_v], add=True)
        return 0

    lax.fori_loop(0, 42, chunk_c, 0)
    plsc.subcore_barrier()
    r1 = pl.multiple_of(sid * 640, 640)
    pltpu.sync_copy(oacc.at[pl.ds(r1, 640)], out_hbm.at[cid, pl.ds(r1, 640)])


def _k4c_body(o1_ref, b1_ref, w2_ref, s2_ref, d2_ref, a2_ref, hw2_ref, mx_ref):
    h1 = jnp.maximum(o1_ref[0] + o1_ref[1] + b1_ref[...], 0.0)
    hw2t = lax.dot_general(w2_ref[...], h1, (((0,), (1,)), ((), ())),
                           preferred_element_type=f32)
    hw2_ref[...] = hw2t
    a2 = jnp.concatenate([hw2t * s2_ref[0, 0], hw2t * d2_ref[0, 0]], axis=0)
    a2_ref[...] = a2
    cur = jnp.max(a2, axis=1, keepdims=True)

    @pl.when(pl.program_id(0) == 0)
    def _():
        mx_ref[...] = cur

    @pl.when(pl.program_id(0) != 0)
    def _():
        mx_ref[...] = jnp.maximum(mx_ref[...], cur)


def _k4c_gat2_lin(o1p, b1, w2, s2, d2):
    return pl.pallas_call(
        _k4c_body,
        grid=(NP // 512,),
        in_specs=[
            pl.BlockSpec((2, 512, FP), lambda i: (0, i, 0)),
            pl.BlockSpec((1, FP), lambda i: (0, 0)),
            pl.BlockSpec((FP, 1), lambda i: (0, 0)),
            pl.BlockSpec((1, 1), lambda i: (0, 0)),
            pl.BlockSpec((1, 1), lambda i: (0, 0)),
        ],
        out_specs=[
            pl.BlockSpec((2, 512), lambda i: (0, i)),
            pl.BlockSpec((1, 512), lambda i: (0, i)),
            pl.BlockSpec((2, 1), lambda i: (0, 0)),
        ],
        out_shape=[
            jax.ShapeDtypeStruct((2, NP), f32),
            jax.ShapeDtypeStruct((1, NP), f32),
            jax.ShapeDtypeStruct((2, 1), f32),
        ],
    )(o1p, b1, w2, s2, d2)


_SC5B = (
    [pltpu.VMEM((16,), f32)]
    + [pltpu.VMEM((256,), i32)] * 2
    + [pltpu.VMEM((512,), i32)]
    + [pltpu.VMEM((512,), f32)]
    + [pltpu.VMEM((256,), f32)] * 3
    + [pltpu.VMEM_SHARED((2 * NP,), f32)]
    + [pltpu.VMEM_SHARED((NP,), f32)] * 3
    + [pltpu.SemaphoreType.DMA]
)


@functools.partial(
    pl.kernel,
    out_type=jax.ShapeDtypeStruct((2 * NP,), f32),
    mesh=_MESH,
    scratch_types=_SC5B,
)
def _k5b_gat2_edges(a2s_hbm, a2d_hbm, hw2_hbm, es_hbm, ed_hbm, c2_hbm, z1_hbm,
                    out_hbm, gv_v, es_v, ed_v, gidx_v, gbuf_v, exb_v, dnc_v,
                    vb_v, tblS, hw2S, dacc, oacc, sem):
    cid = lax.axis_index("c")
    sid = lax.axis_index("s")

    pltpu.sync_copy(c2_hbm, gv_v)

    @pl.when(sid == 0)
    def _():
        pltpu.sync_copy(a2s_hbm, tblS.at[pl.ds(0, NP)])

    @pl.when(sid == 1)
    def _():
        pltpu.sync_copy(a2d_hbm, tblS.at[pl.ds(NP, NP)])

    @pl.when(sid == 2)
    def _():
        pltpu.sync_copy(hw2_hbm, hw2S)

    r0 = pl.multiple_of(sid * 640, 640)
    pltpu.sync_copy(z1_hbm, dacc.at[pl.ds(r0, 640)])
    pltpu.sync_copy(z1_hbm, oacc.at[pl.ds(r0, 640)])
    plsc.subcore_barrier()

    def chunk_a(ci, _):
        e0 = pl.multiple_of(sid * 10752 + ci * 256, 256)
        pltpu.sync_copy(es_hbm.at[pl.ds(e0, 256)], es_v)
        pltpu.sync_copy(ed_hbm.at[pl.ds(e0, 256)], ed_v)

        def bld(g, _):
            sl = pl.ds(g * 16, 16)
            gidx_v[sl] = es_v[sl]
            gidx_v[pl.ds(256 + g * 16, 16)] = NP + ed_v[sl]
            return 0

        lax.fori_loop(0, 16, bld, 0)
        pltpu.async_copy(tblS.at[gidx_v], gbuf_v, sem).wait()

        def group(g, _):
            al = gbuf_v[pl.ds(g * 16, 16)] + gbuf_v[pl.ds(256 + g * 16, 16)]
            exb_v[pl.ds(g * 16, 16)] = jnp.exp(_leaky(al) - gv_v[...])
            return 0

        lax.fori_loop(0, 16, group, 0)
        pltpu.sync_copy(exb_v, dacc.at[ed_v], add=True)
        return 0

    lax.fori_loop(0, 42, chunk_a, 0)
    plsc.subcore_barrier()

    def chunk_c(ci, _):
        e0 = pl.multiple_of((sid * 2 + cid) * 5376 + ci * 256, 256)
        pltpu.sync_copy(es_hbm.at[pl.ds(e0, 256)], es_v)
        pltpu.sync_copy(ed_hbm.at[pl.ds(e0, 256)], ed_v)

        def bld(g, _):
            sl = pl.ds(g * 16, 16)
            gidx_v[sl] = es_v[sl]
            gidx_v[pl.ds(256 + g * 16, 16)] = NP + ed_v[sl]
            return 0

        lax.fori_loop(0, 16, bld, 0)
        pltpu.async_copy(tblS.at[gidx_v], gbuf_v, sem).wait()
        pltpu.async_copy(dacc.at[ed_v], dnc_v, sem).wait()
        pltpu.async_copy(hw2S.at[es_v], vb_v, sem).wait()

        def group(g, _):
            sl = pl.ds(g * 16, 16)
            al = gbuf_v[sl] + gbuf_v[pl.ds(256 + g * 16, 16)]
            ex = jnp.exp(_leaky(al) - gv_v[...])
            vb_v[sl] = vb_v[sl] * ex / (dnc_v[sl] + 1e-16)
            return 0

        lax.fori_loop(0, 16, group, 0)
        pltpu.sync_copy(vb_v, oacc.at[ed_v], add=True)
        return 0

    lax.fori_loop(0, 21, chunk_c, 0)
    plsc.subcore_barrier()
    r1 = pl.multiple_of(sid * 640, 640)
    o1 = pl.multiple_of(cid * NP + sid * 640, 128)
    pltpu.sync_copy(oacc.at[pl.ds(r1, 640)], out_hbm.at[pl.ds(o1, 640)])


def _k6_body(x2_ref, o2_ref, b2_ref, we_ref, be_ref, z_ref, xo_ref):
    att = jax.nn.sigmoid(o2_ref[0] + o2_ref[1] + b2_ref[0, 0])
    emb = jnp.tanh(jnp.dot(x2_ref[...], we_ref[...], preferred_element_type=f32)
                   + be_ref[...])
    z = att * emb
    z_ref[...] = z
    xo_ref[0, ...] = (jnp.max(z, axis=0, keepdims=True)
                      + jnp.sum(z, axis=0, keepdims=True) * (1.0 / L))


def _k6_readout(x2, o2p, b2, wemb, bemb):
    return pl.pallas_call(
        _k6_body,
        grid=(B,),
        in_specs=[
            pl.BlockSpec((L, FP), lambda i: (i, 0)),
            pl.BlockSpec((2, L, 1), lambda i: (0, i, 0)),
            pl.BlockSpec((1, 1), lambda i: (0, 0)),
            pl.BlockSpec((FP, HID), lambda i: (0, 0)),
            pl.BlockSpec((1, HID), lambda i: (0, 0)),
        ],
        out_specs=[
            pl.BlockSpec((L, HID), lambda i: (i, 0)),
            pl.BlockSpec((1, 1, HID), lambda i: (i, 0, 0)),
        ],
        out_shape=[
            jax.ShapeDtypeStruct((N, HID), f32),
            jax.ShapeDtypeStruct((B, 1, HID), f32),
        ],
    )(x2, o2p, b2, wemb, bemb)


def _k7_body(a_ref, b_ref, o_ref):
    acc = lax.dot_general(a_ref[...], b_ref[...],
                          (((1,), (1,)), ((), ())),
                          preferred_element_type=f32)
    o_ref[...] = jax.nn.sigmoid(acc)


def _k7_decode(z):
    return pl.pallas_call(
        _k7_body,
        grid=(10, 10),
        in_specs=[
            pl.BlockSpec((1024, HID), lambda i, j: (i, 0)),
            pl.BlockSpec((1024, HID), lambda i, j: (j, 0)),
        ],
        out_specs=pl.BlockSpec((1024, 1024), lambda i, j: (i, j)),
        out_shape=jax.ShapeDtypeStruct((N, N), f32),
    )(z, z)


_RB = 128
_NBLK = 79
_ACC = _RB * N + 256


@functools.partial(
    pl.kernel,
    out_type=jax.ShapeDtypeStruct((N * N,), f32),
    mesh=_MESH,
    scratch_types=[
        pltpu.VMEM((10240,), i32),
        pltpu.VMEM((10240,), f32),
        pltpu.VMEM((10240,), i32),
        pltpu.VMEM((10240,), f32),
        pltpu.VMEM_SHARED((_ACC,), f32),
        pltpu.SemaphoreType.DMA,
    ],
)
def _k8_adjacency(src_hbm, dst_hbm, attr_hbm, zbig_hbm, out_hbm,
                  fv, av, idxb, valb, acc, sem):
    cid = lax.axis_index("c")
    sid = lax.axis_index("s")
    s0 = pl.multiple_of(sid * 10240, 10240)
    pltpu.sync_copy(src_hbm.at[pl.ds(s0, 10240)], idxb)
    pltpu.sync_copy(dst_hbm.at[pl.ds(s0, 10240)], fv)
    pltpu.sync_copy(attr_hbm.at[pl.ds(s0, 10240)], av)
    z0 = pl.multiple_of(sid * 80000, 128)
    pltpu.sync_copy(zbig_hbm.at[pl.ds(z0, 80000)], acc.at[pl.ds(z0, 80000)])

    def pre(g, _):
        sl = pl.ds(g * 16, 16)
        fv[sl] = idxb[sl] * N + fv[sl]
        return 0

    lax.fori_loop(0, 640, pre, 0)
    plsc.subcore_barrier()

    def build(base):
        def group(g, _):
            sl = pl.ds(g * 16, 16)
            f16 = fv[sl]
            m = (f16 >= base * N) & (f16 < (base + _RB) * N)
            dummy = _RB * N + ((g * 16 + _iota16()) & 255)
            idxb[sl] = jnp.where(m, f16 - base * N, dummy)
            valb[sl] = jnp.where(m, av[sl], 0.0)
            return 0

        lax.fori_loop(0, 640, group, 0)

    def transition(base_a, base_b):
        def group(g, _):
            sl = pl.ds(g * 16, 16)
            f16 = fv[sl]
            ma = (f16 >= base_a * N) & (f16 < (base_a + _RB) * N)
            mb = (f16 >= base_b * N) & (f16 < (base_b + _RB) * N)
            dummy = _RB * N + ((g * 16 + _iota16()) & 255)
            idxb[sl] = jnp.where(mb, f16 - base_b * N,
                                 jnp.where(ma, f16 - base_a * N, dummy))
            valb[sl] = jnp.where(mb, av[sl], jnp.where(ma, -av[sl], 0.0))
            return 0

        lax.fori_loop(0, 640, group, 0)

    build(cid * _RB)
    pltpu.sync_copy(valb, acc.at[idxb], add=True)

    def blk(bi, _):
        b = 2 * bi + cid
        base = b * _RB
        ok = b < _NBLK
        plsc.subcore_barrier()

        @pl.when(ok)
        def _():
            row0 = base + sid * 8

            @pl.when(row0 + 8 <= N)
            def _():
                a0 = pl.multiple_of(sid * (8 * N), 128)
                o0 = pl.multiple_of(row0 * N, 128)
                pltpu.sync_copy(acc.at[pl.ds(a0, 8 * N)],
                                out_hbm.at[pl.ds(o0, 8 * N)])

        plsc.subcore_barrier()

        @pl.when(b + 2 < _NBLK)
        def _():
            transition(base, (b + 2) * _RB)
            pltpu.sync_copy(valb, acc.at[idxb], add=True)

        return 0

    lax.fori_loop(0, 40, blk, 0)


def kernel(x_idx, edge_index, edge_attr, length, embed_table, W_enc, b_enc,
           W_gat1, att_src1, att_dst1, b_gat1, W_gat2, att_src2, att_dst2,
           b_gat2, W_emb, b_emb):
    src = edge_index[0].astype(i32)
    dst = edge_index[1].astype(i32)

    idxp = jnp.concatenate([x_idx.astype(i32), jnp.zeros((NP - N,), i32)])
    efill = jnp.arange(EPAD - E, dtype=i32) % N
    srcp = jnp.concatenate([src, efill])
    dstp = jnp.concatenate([dst, efill])
    attrp = jnp.concatenate([edge_attr.astype(f32), jnp.zeros((EPAD - E,), f32)])
    loops = jnp.arange(N, dtype=i32)
    gfill = jnp.arange(ENP - EN, dtype=i32)
    esg = jnp.concatenate([src, loops, gfill % N])
    edg = jnp.concatenate([dst, loops, N + (gfill % (NP - N))])

    z1 = jnp.zeros((640,), f32)
    z128 = jnp.zeros((640, FP), f32)
    zbig = jnp.zeros((16 * 80000,), f32)

    wencp = jnp.concatenate([W_enc, jnp.zeros((IN_DIM, FP - HID), f32)], axis=1)
    bencp = jnp.concatenate([b_enc, jnp.zeros((FP - HID,), f32)]).reshape(1, FP)
    w1p = jnp.zeros((FP, FP), f32).at[:HID, :F64].set(W_gat1)
    heads_of_col = jnp.arange(FP, dtype=i32) // GATC
    maskh = (heads_of_col[:, None] == jnp.arange(HEADS, dtype=i32)[None, :])
    attcat = jnp.concatenate(
        [jnp.concatenate([att_src1.reshape(-1), jnp.zeros((FP - F64,), f32)])[:, None],
         jnp.concatenate([att_dst1.reshape(-1), jnp.zeros((FP - F64,), f32)])[:, None]],
        axis=1)
    bd = jnp.concatenate([maskh.astype(f32) * attcat[:, :1],
                          maskh.astype(f32) * attcat[:, 1:]], axis=1)
    b1p = jnp.concatenate([b_gat1, jnp.zeros((FP - F64,), f32)]).reshape(1, FP)
    w2p = jnp.concatenate([W_gat2, jnp.zeros((FP - F64, 1), f32)], axis=0)
    wembp = jnp.concatenate([W_emb, jnp.zeros((FP - HID, HID), f32)], axis=0)

    x0 = _k1_embed(embed_table, idxp)
    x1 = _k2_encode(x0, wencp, bencp)
    msgp = _k3_message(x1, srcp, dstp, attrp, z128)
    x2, hw1, av8, mx1 = _k4a_gat1_lin(msgp, w1p, bd)

    m1 = mx1.reshape(8)
    c1 = _leaky(m1[:HEADS] + m1[HEADS:])
    c1b = jnp.broadcast_to(c1[:, None], (HEADS, 16)).reshape(64)
    o1p = _k4b_gat1_edges(av8[0], av8[1], av8[2], av8[3], av8[4], av8[5],
                          av8[6], av8[7], hw1, esg, edg, c1b, z1, z128)

    a2t, hw2t, mx2 = _k4c_gat2_lin(o1p, b1p, w2p,
                                   att_src2.reshape(1, 1), att_dst2.reshape(1, 1))
    c2 = _leaky(mx2[0, 0] + mx2[1, 0])
    c2b = jnp.full((16,), c2, f32)
    o2p = _k5b_gat2_edges(a2t[0], a2t[1], hw2t[0], esg, edg, c2b, z1)

    a_ori = _k8_adjacency(srcp, dstp, attrp, zbig)

    z, xout = _k6_readout(x2, o2p.reshape(2, NP, 1), b_gat2.reshape(1, 1),
                          wembp, b_emb.reshape(1, HID))
    a_pred = _k7_decode(z)

    return (a_pred, xout.reshape(B, HID), a_ori.reshape(N, N))


# speedup vs baseline: 1.1246x; 1.0310x over previous
import functools

import jax
import jax.numpy as jnp
from jax import lax
from jax.experimental import pallas as pl
from jax.experimental.pallas import tpu as pltpu
import jax.experimental.pallas.tpu_sc as plsc

N = 10000
NP = 10240
E = 160000
EPAD = 163840
EN = 170000
ENP = 172032
IN_DIM = 256
HID = 96
HEADS = 4
GATC = 16
F64 = HEADS * GATC
FP = 128
B = 50
L = 200

f32 = jnp.float32
i32 = jnp.int32

_MESH = plsc.VectorSubcoreMesh(core_axis_name="c", subcore_axis_name="s")


def _iota16():
    return lax.iota(i32, 16)


def _leaky(x):
    return jnp.where(x >= 0, x, 0.2 * x)


@functools.partial(
    pl.kernel,
    out_type=jax.ShapeDtypeStruct((NP, IN_DIM), f32),
    mesh=_MESH,
    scratch_types=[
        pltpu.VMEM((4, 80), i32),
        pltpu.VMEM((320, IN_DIM), f32),
        pltpu.SemaphoreType.DMA,
    ],
)
def _k1_embed(table_hbm, idx_hbm, out_hbm, idx_v, rows_v, sem):
    wid = lax.axis_index("s") * 2 + lax.axis_index("c")
    base = pl.multiple_of(wid * 320, 320)
    for t in range(4):
        pltpu.sync_copy(idx_hbm.at[pl.ds(base + t * 80, 80)], idx_v.at[t])
    for t in range(4):
        pltpu.async_copy(table_hbm.at[idx_v.at[t]],
                         rows_v.at[pl.ds(t * 80, 80)], sem).wait()
    pltpu.sync_copy(rows_v, out_hbm.at[pl.ds(base, 320)])


def _k2_body(x_ref, w_ref, b_ref, o_ref):
    acc = jnp.dot(x_ref[...], w_ref[...], preferred_element_type=f32)
    o_ref[...] = jnp.tanh(acc + b_ref[...])


def _k2_encode(x0, w, b):
    return pl.pallas_call(
        _k2_body,
        grid=(NP // 512,),
        in_specs=[
            pl.BlockSpec((512, IN_DIM), lambda i: (i, 0)),
            pl.BlockSpec((IN_DIM, FP), lambda i: (0, 0)),
            pl.BlockSpec((1, FP), lambda i: (0, 0)),
        ],
        out_specs=pl.BlockSpec((512, FP), lambda i: (i, 0)),
        out_shape=jax.ShapeDtypeStruct((NP, FP), f32),
    )(x0, w, b)


@functools.partial(
    pl.kernel,
    out_type=jax.ShapeDtypeStruct((2, NP, FP), f32),
    mesh=_MESH,
    scratch_types=[
        pltpu.VMEM((128,), i32),
        pltpu.VMEM((128,), i32),
        pltpu.VMEM((144,), f32),
        pltpu.VMEM((128, FP), f32),
        pltpu.VMEM((128, FP), f32),
        pltpu.VMEM_SHARED((NP, FP), f32),
        pltpu.SemaphoreType.DMA,
    ],
)
def _k3_message(x1_hbm, src_hbm, dst_hbm, attr_hbm, z128_hbm, out_hbm,
                es_v, ed_v, at_v, rows_v, sc_v, acc, sem):
    cid = lax.axis_index("c")
    sid = lax.axis_index("s")
    wid = sid * 2 + cid
    pltpu.sync_copy(z128_hbm, acc.at[pl.ds(pl.multiple_of(sid * 640, 640), 640)])
    plsc.subcore_barrier()

    def chunk(ci, _):
        e0 = pl.multiple_of(wid * 5120 + ci * 128, 128)
        pltpu.sync_copy(src_hbm.at[pl.ds(e0, 128)], es_v)
        pltpu.sync_copy(dst_hbm.at[pl.ds(e0, 128)], ed_v)
        pltpu.sync_copy(attr_hbm.at[pl.ds(e0, 128)], at_v.at[pl.ds(0, 128)])
        pltpu.async_copy(x1_hbm.at[es_v], rows_v, sem).wait()

        def edge(j, _):
            aj = at_v[pl.ds(j, 16)][0]
            bc = jnp.full((16,), aj, f32)
            for c in range(8):
                sc_v[j, pl.ds(c * 16, 16)] = rows_v[j, pl.ds(c * 16, 16)] * bc
            return 0

        lax.fori_loop(0, 128, edge, 0)
        pltpu.sync_copy(sc_v, acc.at[ed_v], add=True)
        return 0

    lax.fori_loop(0, 40, chunk, 0)
    plsc.subcore_barrier()
    r0 = pl.multiple_of(sid * 640, 640)
    pltpu.sync_copy(acc.at[pl.ds(r0, 640)], out_hbm.at[cid, pl.ds(r0, 640)])


def _k4a_body(p_ref, w_ref, bd_ref, x2_ref, hw_ref, av8_ref, mx_ref):
    x2 = p_ref[0] + p_ref[1]
    x2_ref[...] = x2
    hw = jnp.dot(x2, w_ref[...], preferred_element_type=f32)
    hw_ref[...] = hw
    av8 = lax.dot_general(bd_ref[...], hw, (((0,), (1,)), ((), ())),
                          preferred_element_type=f32)
    av8_ref[...] = av8
    cur = jnp.max(av8, axis=1, keepdims=True)

    @pl.when(pl.program_id(0) == 0)
    def _():
        mx_ref[...] = cur

    @pl.when(pl.program_id(0) != 0)
    def _():
        mx_ref[...] = jnp.maximum(mx_ref[...], cur)


def _k4a_gat1_lin(msgp, w1, bd):
    return pl.pallas_call(
        _k4a_body,
        grid=(NP // 512,),
        in_specs=[
            pl.BlockSpec((2, 512, FP), lambda i: (0, i, 0)),
            pl.BlockSpec((FP, FP), lambda i: (0, 0)),
            pl.BlockSpec((FP, 2 * HEADS), lambda i: (0, 0)),
        ],
        out_specs=[
            pl.BlockSpec((512, FP), lambda i: (i, 0)),
            pl.BlockSpec((512, FP), lambda i: (i, 0)),
            pl.BlockSpec((2 * HEADS, 512), lambda i: (0, i)),
            pl.BlockSpec((2 * HEADS, 1), lambda i: (0, 0)),
        ],
        out_shape=[
            jax.ShapeDtypeStruct((NP, FP), f32),
            jax.ShapeDtypeStruct((NP, FP), f32),
            jax.ShapeDtypeStruct((2 * HEADS, NP), f32),
            jax.ShapeDtypeStruct((2 * HEADS, 1), f32),
        ],
    )(msgp, w1, bd)


_SC4A = (
    [pltpu.VMEM((64,), f32)]
    + [pltpu.VMEM((256,), i32)] * 2
    + [pltpu.VMEM((2048,), i32)]
    + [pltpu.VMEM((2048,), f32)]
    + [pltpu.VMEM((1024,), i32)]
    + [pltpu.VMEM((1024,), f32)]
    + [pltpu.VMEM_SHARED((8 * NP,), f32)]
    + [pltpu.VMEM_SHARED((4 * NP,), f32)]
    + [pltpu.SemaphoreType.DMA]
)


@functools.partial(
    pl.kernel,
    out_type=jax.ShapeDtypeStruct((2, 4 * NP), f32),
    mesh=_MESH,
    scratch_types=_SC4A,
)
def _k4b_gat1_dacc(a0_hbm, a1_hbm, a2_hbm, a3_hbm, a4_hbm, a5_hbm, a6_hbm,
                   a7_hbm, es_hbm, ed_hbm, c1_hbm, z1_hbm, out_hbm,
                   gv_v, esa_v, eda_v, gidx_v, gbuf_v, didx_v, exb_v,
                   tblA, daccA, sem):
    cid = lax.axis_index("c")
    sid = lax.axis_index("s")

    av_in = [a0_hbm, a1_hbm, a2_hbm, a3_hbm, a4_hbm, a5_hbm, a6_hbm, a7_hbm]
    pltpu.sync_copy(c1_hbm, gv_v)
    for h in range(8):
        @pl.when(sid == h)
        def _(h=h):
            pltpu.sync_copy(av_in[h], tblA.at[pl.ds(h * NP, NP)])
    for q in range(4):
        pltpu.sync_copy(z1_hbm, daccA.at[pl.ds(sid * 2560 + q * 640, 640)])
    plsc.subcore_barrier()

    def chunk_a(ci, _):
        e0 = pl.multiple_of((sid * 2 + cid) * 5376 + ci * 256, 256)
        pltpu.sync_copy(es_hbm.at[pl.ds(e0, 256)], esa_v)
        pltpu.sync_copy(ed_hbm.at[pl.ds(e0, 256)], eda_v)

        def bld(g, _):
            s16 = esa_v[pl.ds(g * 16, 16)]
            d16 = eda_v[pl.ds(g * 16, 16)]
            for h in range(4):
                gidx_v[pl.ds(h * 256 + g * 16, 16)] = h * NP + s16
                gidx_v[pl.ds((4 + h) * 256 + g * 16, 16)] = (4 + h) * NP + d16
                didx_v[pl.ds(h * 256 + g * 16, 16)] = h * NP + d16
            return 0

        lax.fori_loop(0, 16, bld, 0)
        pltpu.async_copy(tblA.at[gidx_v], gbuf_v, sem).wait()

        def group(g, _):
            for h in range(4):
                al = (gbuf_v[pl.ds(h * 256 + g * 16, 16)]
                      + gbuf_v[pl.ds((4 + h) * 256 + g * 16, 16)])
                exb_v[pl.ds(h * 256 + g * 16, 16)] = jnp.exp(
                    _leaky(al) - gv_v[pl.ds(h * 16, 16)])
            return 0

        lax.fori_loop(0, 16, group, 0)
        pltpu.async_copy(exb_v, daccA.at[didx_v], sem, add=True).wait()
        return 0

    lax.fori_loop(0, 21, chunk_a, 0)
    plsc.subcore_barrier()
    pltpu.sync_copy(daccA.at[pl.ds(sid * 2560, 2560)],
                    out_hbm.at[cid, pl.ds(sid * 2560, 2560)])


_SC4B = (
    [pltpu.VMEM((64,), f32)]
    + [pltpu.VMEM((128,), i32)] * 2
    + [pltpu.VMEM((2048,), i32)]
    + [pltpu.VMEM((2048,), f32)]
    + [pltpu.VMEM((1024,), i32)]
    + [pltpu.VMEM((1024,), f32)]
    + [pltpu.VMEM((512,), f32)] * 2
    + [pltpu.VMEM((144,), f32)] * 4
    + [pltpu.VMEM((128, FP), f32)] * 2
    + [pltpu.VMEM_SHARED((8 * NP,), f32)]
    + [pltpu.VMEM_SHARED((4 * NP,), f32)]
    + [pltpu.VMEM_SHARED((NP, FP), f32)]
    + [pltpu.SemaphoreType.DMA]
)


@functools.partial(
    pl.kernel,
    out_type=jax.ShapeDtypeStruct((2, NP, FP), f32),
    mesh=_MESH,
    scratch_types=_SC4B,
)
def _k4b_gat1_edges(a0_hbm, a1_hbm, a2_hbm, a3_hbm, a4_hbm, a5_hbm, a6_hbm,
                    a7_hbm, hw_hbm, es_hbm, ed_hbm, c1_hbm, dacc_hbm, z128_hbm,
                    out_hbm, gv_v, esc_v, edc_v, gidx_v, gbuf_v,
                    didx_v, exb_v, tmp0_v, tmp1_v, cf0, cf1, cf2, cf3,
                    hwr_v, sc_v, tblA, daccA, oacc, sem):
    cid = lax.axis_index("c")
    sid = lax.axis_index("s")
    cf = [cf0, cf1, cf2, cf3]

    av_in = [a0_hbm, a1_hbm, a2_hbm, a3_hbm, a4_hbm, a5_hbm, a6_hbm, a7_hbm]
    pltpu.sync_copy(c1_hbm, gv_v)
    for h in range(8):
        @pl.when(sid == h)
        def _(h=h):
            pltpu.sync_copy(av_in[h], tblA.at[pl.ds(h * NP, NP)])
    r0 = pl.multiple_of(sid * 640, 640)
    pltpu.sync_copy(z128_hbm, oacc.at[pl.ds(r0, 640)])

    for q in range(5):
        d0 = pl.multiple_of(sid * 2560 + q * 512, 512)
        pltpu.sync_copy(dacc_hbm.at[0, pl.ds(d0, 512)], tmp0_v)
        pltpu.sync_copy(dacc_hbm.at[1, pl.ds(d0, 512)], tmp1_v)

        def mrg(g, _):
            sl = pl.ds(g * 16, 16)
            tmp0_v[sl] = tmp0_v[sl] + tmp1_v[sl]
            return 0

        lax.fori_loop(0, 32, mrg, 0)
        pltpu.sync_copy(tmp0_v, daccA.at[pl.ds(d0, 512)])

    def zrow(j, _):
        for c in range(4):
            sc_v[j, pl.ds(F64 + c * 16, 16)] = jnp.zeros((16,), f32)
        return 0

    lax.fori_loop(0, 128, zrow, 0)
    plsc.subcore_barrier()

    def chunk_c(ci, _):
        e0 = pl.multiple_of((sid * 2 + cid) * 5376 + ci * 128, 128)
        pltpu.sync_copy(es_hbm.at[pl.ds(e0, 128)], esc_v)
        pltpu.sync_copy(ed_hbm.at[pl.ds(e0, 128)], edc_v)
        pltpu.async_copy(hw_hbm.at[esc_v], hwr_v, sem).wait()

        def bld(g, _):
            s16 = esc_v[pl.ds(g * 16, 16)]
            d16 = edc_v[pl.ds(g * 16, 16)]
            for h in range(4):
                gidx_v[pl.ds(h * 128 + g * 16, 16)] = h * NP + s16
                gidx_v[pl.ds((4 + h) * 128 + g * 16, 16)] = (4 + h) * NP + d16
                didx_v[pl.ds(h * 128 + g * 16, 16)] = h * NP + d16
            return 0

        lax.fori_loop(0, 8, bld, 0)
        pltpu.async_copy(tblA.at[gidx_v], gbuf_v, sem).wait()
        pltpu.async_copy(daccA.at[didx_v], exb_v, sem).wait()

        def group(g, _):
            for h in range(4):
                al = (gbuf_v[pl.ds(h * 128 + g * 16, 16)]
                      + gbuf_v[pl.ds((4 + h) * 128 + g * 16, 16)])
                ex = jnp.exp(_leaky(al) - gv_v[pl.ds(h * 16, 16)])
                den = exb_v[pl.ds(h * 128 + g * 16, 16)]
                cf[h][pl.ds(g * 16, 16)] = ex / (den + 1e-16)
            return 0

        lax.fori_loop(0, 8, group, 0)

        def edge(j, _):
            for h in range(4):
                cj = cf[h][pl.ds(j, 16)][0]
                bc = jnp.full((16,), cj, f32)
                sc_v[j, pl.ds(h * 16, 16)] = hwr_v[j, pl.ds(h * 16, 16)] * bc
            return 0

        lax.fori_loop(0, 128, edge, 0)
        pltpu.sync_copy(sc_v, oacc.at[edc_v], add=True)
        return 0

    lax.fori_loop(0, 42, chunk_c, 0)
    plsc.subcore_barrier()
    r1 = pl.multiple_of(sid * 640, 640)
    pltpu.sync_copy(oacc.at[pl.ds(r1, 640)], out_hbm.at[cid, pl.ds(r1, 640)])


def _k4c_body(o1_ref, b1_ref, w2_ref, s2_ref, d2_ref, a2_ref, hw2_ref, mx_ref):
    h1 = jnp.maximum(o1_ref[0] + o1_ref[1] + b1_ref[...], 0.0)
    hw2t = lax.dot_general(w2_ref[...], h1, (((0,), (1,)), ((), ())),
                           preferred_element_type=f32)
    hw2_ref[...] = hw2t
    a2 = jnp.concatenate([hw2t * s2_ref[0, 0], hw2t * d2_ref[0, 0]], axis=0)
    a2_ref[...] = a2
    cur = jnp.max(a2, axis=1, keepdims=True)

    @pl.when(pl.program_id(0) == 0)
    def _():
        mx_ref[...] = cur

    @pl.when(pl.program_id(0) != 0)
    def _():
        mx_ref[...] = jnp.maximum(mx_ref[...], cur)


def _k4c_gat2_lin(o1p, b1, w2, s2, d2):
    return pl.pallas_call(
        _k4c_body,
        grid=(NP // 512,),
        in_specs=[
            pl.BlockSpec((2, 512, FP), lambda i: (0, i, 0)),
            pl.BlockSpec((1, FP), lambda i: (0, 0)),
            pl.BlockSpec((FP, 1), lambda i: (0, 0)),
            pl.BlockSpec((1, 1), lambda i: (0, 0)),
            pl.BlockSpec((1, 1), lambda i: (0, 0)),
        ],
        out_specs=[
            pl.BlockSpec((2, 512), lambda i: (0, i)),
            pl.BlockSpec((1, 512), lambda i: (0, i)),
            pl.BlockSpec((2, 1), lambda i: (0, 0)),
        ],
        out_shape=[
            jax.ShapeDtypeStruct((2, NP), f32),
            jax.ShapeDtypeStruct((1, NP), f32),
            jax.ShapeDtypeStruct((2, 1), f32),
        ],
    )(o1p, b1, w2, s2, d2)


_SC5B = (
    [pltpu.VMEM((16,), f32)]
    + [pltpu.VMEM((256,), i32)] * 2
    + [pltpu.VMEM((512,), i32)]
    + [pltpu.VMEM((512,), f32)]
    + [pltpu.VMEM((256,), f32)] * 3
    + [pltpu.VMEM_SHARED((2 * NP,), f32)]
    + [pltpu.VMEM_SHARED((NP,), f32)] * 3
    + [pltpu.SemaphoreType.DMA]
)


@functools.partial(
    pl.kernel,
    out_type=jax.ShapeDtypeStruct((2 * NP,), f32),
    mesh=_MESH,
    scratch_types=_SC5B,
)
def _k5b_gat2_edges(a2s_hbm, a2d_hbm, hw2_hbm, es_hbm, ed_hbm, c2_hbm, z1_hbm,
                    out_hbm, gv_v, es_v, ed_v, gidx_v, gbuf_v, exb_v, dnc_v,
                    vb_v, tblS, hw2S, dacc, oacc, sem):
    cid = lax.axis_index("c")
    sid = lax.axis_index("s")

    pltpu.sync_copy(c2_hbm, gv_v)

    @pl.when(sid == 0)
    def _():
        pltpu.sync_copy(a2s_hbm, tblS.at[pl.ds(0, NP)])

    @pl.when(sid == 1)
    def _():
        pltpu.sync_copy(a2d_hbm, tblS.at[pl.ds(NP, NP)])

    @pl.when(sid == 2)
    def _():
        pltpu.sync_copy(hw2_hbm, hw2S)

    r0 = pl.multiple_of(sid * 640, 640)
    pltpu.sync_copy(z1_hbm, dacc.at[pl.ds(r0, 640)])
    pltpu.sync_copy(z1_hbm, oacc.at[pl.ds(r0, 640)])
    plsc.subcore_barrier()

    def chunk_a(ci, _):
        e0 = pl.multiple_of(sid * 10752 + ci * 256, 256)
        pltpu.sync_copy(es_hbm.at[pl.ds(e0, 256)], es_v)
        pltpu.sync_copy(ed_hbm.at[pl.ds(e0, 256)], ed_v)

        def bld(g, _):
            sl = pl.ds(g * 16, 16)
            gidx_v[sl] = es_v[sl]
            gidx_v[pl.ds(256 + g * 16, 16)] = NP + ed_v[sl]
            return 0

        lax.fori_loop(0, 16, bld, 0)
        pltpu.async_copy(tblS.at[gidx_v], gbuf_v, sem).wait()

        def group(g, _):
            al = gbuf_v[pl.ds(g * 16, 16)] + gbuf_v[pl.ds(256 + g * 16, 16)]
            exb_v[pl.ds(g * 16, 16)] = jnp.exp(_leaky(al) - gv_v[...])
            return 0

        lax.fori_loop(0, 16, group, 0)
        pltpu.sync_copy(exb_v, dacc.at[ed_v], add=True)
        return 0

    lax.fori_loop(0, 42, chunk_a, 0)
    plsc.subcore_barrier()

    def chunk_c(ci, _):
        e0 = pl.multiple_of((sid * 2 + cid) * 5376 + ci * 256, 256)
        pltpu.sync_copy(es_hbm.at[pl.ds(e0, 256)], es_v)
        pltpu.sync_copy(ed_hbm.at[pl.ds(e0, 256)], ed_v)

        def bld(g, _):
            sl = pl.ds(g * 16, 16)
            gidx_v[sl] = es_v[sl]
            gidx_v[pl.ds(256 + g * 16, 16)] = NP + ed_v[sl]
            return 0

        lax.fori_loop(0, 16, bld, 0)
        pltpu.async_copy(tblS.at[gidx_v], gbuf_v, sem).wait()
        pltpu.async_copy(dacc.at[ed_v], dnc_v, sem).wait()
        pltpu.async_copy(hw2S.at[es_v], vb_v, sem).wait()

        def group(g, _):
            sl = pl.ds(g * 16, 16)
            al = gbuf_v[sl] + gbuf_v[pl.ds(256 + g * 16, 16)]
            ex = jnp.exp(_leaky(al) - gv_v[...])
            vb_v[sl] = vb_v[sl] * ex / (dnc_v[sl] + 1e-16)
            return 0

        lax.fori_loop(0, 16, group, 0)
        pltpu.sync_copy(vb_v, oacc.at[ed_v], add=True)
        return 0

    lax.fori_loop(0, 21, chunk_c, 0)
    plsc.subcore_barrier()
    r1 = pl.multiple_of(sid * 640, 640)
    o1 = pl.multiple_of(cid * NP + sid * 640, 128)
    pltpu.sync_copy(oacc.at[pl.ds(r1, 640)], out_hbm.at[pl.ds(o1, 640)])


def _k6_body(x2_ref, o2_ref, b2_ref, we_ref, be_ref, z_ref, xo_ref):
    att = jax.nn.sigmoid(o2_ref[0] + o2_ref[1] + b2_ref[0, 0])
    emb = jnp.tanh(jnp.dot(x2_ref[...], we_ref[...], preferred_element_type=f32)
                   + be_ref[...])
    z = att * emb
    z_ref[...] = z
    xo_ref[0, ...] = (jnp.max(z, axis=0, keepdims=True)
                      + jnp.sum(z, axis=0, keepdims=True) * (1.0 / L))


def _k6_readout(x2, o2p, b2, wemb, bemb):
    return pl.pallas_call(
        _k6_body,
        grid=(B,),
        in_specs=[
            pl.BlockSpec((L, FP), lambda i: (i, 0)),
            pl.BlockSpec((2, L, 1), lambda i: (0, i, 0)),
            pl.BlockSpec((1, 1), lambda i: (0, 0)),
            pl.BlockSpec((FP, HID), lambda i: (0, 0)),
            pl.BlockSpec((1, HID), lambda i: (0, 0)),
        ],
        out_specs=[
            pl.BlockSpec((L, HID), lambda i: (i, 0)),
            pl.BlockSpec((1, 1, HID), lambda i: (i, 0, 0)),
        ],
        out_shape=[
            jax.ShapeDtypeStruct((N, HID), f32),
            jax.ShapeDtypeStruct((B, 1, HID), f32),
        ],
    )(x2, o2p, b2, wemb, bemb)


def _k7_body(a_ref, b_ref, o_ref):
    acc = lax.dot_general(a_ref[...], b_ref[...],
                          (((1,), (1,)), ((), ())),
                          preferred_element_type=f32)
    o_ref[...] = jax.nn.sigmoid(acc)


def _k7_decode(z):
    return pl.pallas_call(
        _k7_body,
        grid=(10, 10),
        in_specs=[
            pl.BlockSpec((1024, HID), lambda i, j: (i, 0)),
            pl.BlockSpec((1024, HID), lambda i, j: (j, 0)),
        ],
        out_specs=pl.BlockSpec((1024, 1024), lambda i, j: (i, j)),
        out_shape=jax.ShapeDtypeStruct((N, N), f32),
    )(z, z)


_RB = 128
_NBLK = 79
_ACC = _RB * N + 256


@functools.partial(
    pl.kernel,
    out_type=jax.ShapeDtypeStruct((N * N,), f32),
    mesh=_MESH,
    scratch_types=[
        pltpu.VMEM((10240,), i32),
        pltpu.VMEM((10240,), f32),
        pltpu.VMEM((10240,), i32),
        pltpu.VMEM((10240,), f32),
        pltpu.VMEM_SHARED((_ACC,), f32),
        pltpu.SemaphoreType.DMA,
    ],
)
def _k8_adjacency(src_hbm, dst_hbm, attr_hbm, zbig_hbm, out_hbm,
                  fv, av, idxb, valb, acc, sem):
    cid = lax.axis_index("c")
    sid = lax.axis_index("s")
    s0 = pl.multiple_of(sid * 10240, 10240)
    pltpu.sync_copy(src_hbm.at[pl.ds(s0, 10240)], idxb)
    pltpu.sync_copy(dst_hbm.at[pl.ds(s0, 10240)], fv)
    pltpu.sync_copy(attr_hbm.at[pl.ds(s0, 10240)], av)
    z0 = pl.multiple_of(sid * 80000, 128)
    pltpu.sync_copy(zbig_hbm.at[pl.ds(z0, 80000)], acc.at[pl.ds(z0, 80000)])

    def pre(g, _):
        sl = pl.ds(g * 16, 16)
        fv[sl] = idxb[sl] * N + fv[sl]
        return 0

    lax.fori_loop(0, 640, pre, 0)
    plsc.subcore_barrier()

    def build(base):
        def group(g, _):
            sl = pl.ds(g * 16, 16)
            f16 = fv[sl]
            m = (f16 >= base * N) & (f16 < (base + _RB) * N)
            dummy = _RB * N + ((g * 16 + _iota16()) & 255)
            idxb[sl] = jnp.where(m, f16 - base * N, dummy)
            valb[sl] = jnp.where(m, av[sl], 0.0)
            return 0

        lax.fori_loop(0, 640, group, 0)

    def transition(base_a, base_b):
        def group(g, _):
            sl = pl.ds(g * 16, 16)
            f16 = fv[sl]
            ma = (f16 >= base_a * N) & (f16 < (base_a + _RB) * N)
            mb = (f16 >= base_b * N) & (f16 < (base_b + _RB) * N)
            dummy = _RB * N + ((g * 16 + _iota16()) & 255)
            idxb[sl] = jnp.where(mb, f16 - base_b * N,
                                 jnp.where(ma, f16 - base_a * N, dummy))
            valb[sl] = jnp.where(mb, av[sl], jnp.where(ma, -av[sl], 0.0))
            return 0

        lax.fori_loop(0, 640, group, 0)

    build(cid * _RB)
    pltpu.sync_copy(valb, acc.at[idxb], add=True)

    def blk(bi, _):
        b = 2 * bi + cid
        base = b * _RB
        ok = b < _NBLK
        plsc.subcore_barrier()

        @pl.when(ok)
        def _():
            row0 = base + sid * 8

            @pl.when(row0 + 8 <= N)
            def _():
                a0 = pl.multiple_of(sid * (8 * N), 128)
                o0 = pl.multiple_of(row0 * N, 128)
                pltpu.sync_copy(acc.at[pl.ds(a0, 8 * N)],
                                out_hbm.at[pl.ds(o0, 8 * N)])

        plsc.subcore_barrier()

        @pl.when(b + 2 < _NBLK)
        def _():
            transition(base, (b + 2) * _RB)
            pltpu.sync_copy(valb, acc.at[idxb], add=True)

        return 0

    lax.fori_loop(0, 40, blk, 0)


def kernel(x_idx, edge_index, edge_attr, length, embed_table, W_enc, b_enc,
           W_gat1, att_src1, att_dst1, b_gat1, W_gat2, att_src2, att_dst2,
           b_gat2, W_emb, b_emb):
    src = edge_index[0].astype(i32)
    dst = edge_index[1].astype(i32)

    idxp = jnp.concatenate([x_idx.astype(i32), jnp.zeros((NP - N,), i32)])
    efill = jnp.arange(EPAD - E, dtype=i32) % N
    srcp = jnp.concatenate([src, efill])
    dstp = jnp.concatenate([dst, efill])
    attrp = jnp.concatenate([edge_attr.astype(f32), jnp.zeros((EPAD - E,), f32)])
    loops = jnp.arange(N, dtype=i32)
    gfill = jnp.arange(ENP - EN, dtype=i32)
    esg = jnp.concatenate([src, loops, gfill % N])
    edg = jnp.concatenate([dst, loops, N + (gfill % (NP - N))])

    z1 = jnp.zeros((640,), f32)
    z128 = jnp.zeros((640, FP), f32)
    zbig = jnp.zeros((16 * 80000,), f32)

    wencp = jnp.concatenate([W_enc, jnp.zeros((IN_DIM, FP - HID), f32)], axis=1)
    bencp = jnp.concatenate([b_enc, jnp.zeros((FP - HID,), f32)]).reshape(1, FP)
    w1p = jnp.zeros((FP, FP), f32).at[:HID, :F64].set(W_gat1)
    heads_of_col = jnp.arange(FP, dtype=i32) // GATC
    maskh = (heads_of_col[:, None] == jnp.arange(HEADS, dtype=i32)[None, :])
    attcat = jnp.concatenate(
        [jnp.concatenate([att_src1.reshape(-1), jnp.zeros((FP - F64,), f32)])[:, None],
         jnp.concatenate([att_dst1.reshape(-1), jnp.zeros((FP - F64,), f32)])[:, None]],
        axis=1)
    bd = jnp.concatenate([maskh.astype(f32) * attcat[:, :1],
                          maskh.astype(f32) * attcat[:, 1:]], axis=1)
    b1p = jnp.concatenate([b_gat1, jnp.zeros((FP - F64,), f32)]).reshape(1, FP)
    w2p = jnp.concatenate([W_gat2, jnp.zeros((FP - F64, 1), f32)], axis=0)
    wembp = jnp.concatenate([W_emb, jnp.zeros((FP - HID, HID), f32)], axis=0)

    x0 = _k1_embed(embed_table, idxp)
    x1 = _k2_encode(x0, wencp, bencp)
    msgp = _k3_message(x1, srcp, dstp, attrp, z128)
    x2, hw1, av8, mx1 = _k4a_gat1_lin(msgp, w1p, bd)

    m1 = mx1.reshape(8)
    c1 = _leaky(m1[:HEADS] + m1[HEADS:])
    c1b = jnp.broadcast_to(c1[:, None], (HEADS, 16)).reshape(64)
    daccp = _k4b_gat1_dacc(av8[0], av8[1], av8[2], av8[3], av8[4], av8[5],
                           av8[6], av8[7], esg, edg, c1b, z1)
    o1p = _k4b_gat1_edges(av8[0], av8[1], av8[2], av8[3], av8[4], av8[5],
                          av8[6], av8[7], hw1, esg, edg, c1b, daccp, z128)

    a2t, hw2t, mx2 = _k4c_gat2_lin(o1p, b1p, w2p,
                                   att_src2.reshape(1, 1), att_dst2.reshape(1, 1))
    c2 = _leaky(mx2[0, 0] + mx2[1, 0])
    c2b = jnp.full((16,), c2, f32)
    o2p = _k5b_gat2_edges(a2t[0], a2t[1], hw2t[0], esg, edg, c2b, z1)

    a_ori = _k8_adjacency(srcp, dstp, attrp, zbig)

    z, xout = _k6_readout(x2, o2p.reshape(2, NP, 1), b_gat2.reshape(1, 1),
                          wembp, b_emb.reshape(1, HID))
    a_pred = _k7_decode(z)

    return (a_pred, xout.reshape(B, HID), a_ori.reshape(N, N))
